# baseline (speedup 1.0000x reference)
import functools

import jax
import jax.numpy as jnp
from jax import lax
from jax.experimental import pallas as pl
from jax.experimental.pallas import tpu as pltpu
from jax.experimental.pallas import tpu_sc as plsc

_INFO = plsc.get_sparse_core_info()
_NC = _INFO.num_cores
_NS = _INFO.num_subcores
_NW = _NC * _NS


@functools.partial(jax.jit, static_argnames=("chunk",))
def _gather_rows(table, idx, chunk=512):
    B = idx.shape[0]
    V, D = table.shape
    b_per_w = B // _NW
    n_chunks = b_per_w // chunk
    mesh = plsc.VectorSubcoreMesh(core_axis_name="c", subcore_axis_name="s")

    @functools.partial(
        pl.kernel,
        mesh=mesh,
        out_type=jax.ShapeDtypeStruct((B, D), jnp.float32),
        scratch_types=[
            pltpu.VMEM((chunk,), jnp.int32),
            pltpu.VMEM((chunk, D), jnp.float32),
            pltpu.SemaphoreType.DMA,
        ],
        compiler_params=pltpu.CompilerParams(use_tc_tiling_on_sc=False),
    )
    def k(table_hbm, idx_hbm, out_hbm, idx_v, rows_v, sem):
        wid = lax.axis_index("s") * _NC + lax.axis_index("c")
        w_base = wid * b_per_w

        def body(g, _):
            base = w_base + g * chunk
            pltpu.sync_copy(idx_hbm.at[pl.ds(base, chunk)], idx_v)
            pltpu.async_copy(table_hbm.at[idx_v], rows_v, sem).wait()
            pltpu.sync_copy(rows_v, out_hbm.at[pl.ds(base, chunk)])
            return 0

        lax.fori_loop(0, n_chunks, body, 0)

    return k(table, idx)


def kernel(x, table):
    B0, B1 = x.shape
    D = table.shape[1]
    flat = x.reshape(B0 * B1).astype(jnp.int32)
    rows = _gather_rows(table, flat)
    return rows.reshape(B0, B1, D)


# speedup vs baseline: 1.0638x; 1.0638x over previous
import functools

import jax
import jax.numpy as jnp
from jax import lax
from jax.experimental import pallas as pl
from jax.experimental.pallas import tpu as pltpu
from jax.experimental.pallas import tpu_sc as plsc

_INFO = plsc.get_sparse_core_info()
_NC = _INFO.num_cores
_NS = _INFO.num_subcores
_NW = _NC * _NS


@functools.partial(jax.jit, static_argnames=("chunk",))
def _gather_rows(table, idx, chunk=800):
    B = idx.shape[0]
    V, D = table.shape
    b_per_w = B // _NW
    n_chunks = b_per_w // chunk
    n_pairs = n_chunks // 2
    mesh = plsc.VectorSubcoreMesh(core_axis_name="c", subcore_axis_name="s")

    @functools.partial(
        pl.kernel,
        mesh=mesh,
        out_type=jax.ShapeDtypeStruct((B, D), jnp.float32),
        scratch_types=[
            pltpu.VMEM((2, chunk), jnp.int32),
            pltpu.VMEM((2, chunk, D), jnp.float32),
            pltpu.SemaphoreType.DMA,
            pltpu.SemaphoreType.DMA,
            pltpu.SemaphoreType.DMA,
            pltpu.SemaphoreType.DMA,
        ],
        compiler_params=pltpu.CompilerParams(use_tc_tiling_on_sc=False),
    )
    def k(table_hbm, idx_hbm, out_hbm, idx_v, rows_v, g0, g1, w0, w1):
        wid = lax.axis_index("s") * _NC + lax.axis_index("c")
        w_base = wid * b_per_w
        gsem = (g0, g1)
        wsem = (w0, w1)

        def idx_src(g):
            return idx_hbm.at[pl.ds(w_base + g * chunk, chunk)]

        def out_dst(g):
            return out_hbm.at[pl.ds(w_base + g * chunk, chunk)]

        def gather(g, b):
            pltpu.async_copy(table_hbm.at[idx_v.at[b]], rows_v.at[b], gsem[b])

        def gather_wait(b):
            pltpu.make_async_copy(
                table_hbm.at[idx_v.at[b]], rows_v.at[b], gsem[b]).wait()

        def write(g, b):
            pltpu.async_copy(rows_v.at[b], out_dst(g), wsem[b])

        def write_wait(g, b):
            pltpu.make_async_copy(rows_v.at[b], out_dst(g), wsem[b]).wait()

        pltpu.sync_copy(idx_src(0), idx_v.at[0])
        gather(0, 0)

        def pair_body(p, _):
            for b in (0, 1):
                g = 2 * p + b
                nb = 1 - b
                if b == 1:
                    write_wait(g - 1, nb)
                else:
                    @pl.when(p > 0)
                    def _():
                        write_wait(g - 1, nb)
                if b == 0:
                    pltpu.sync_copy(idx_src(g + 1), idx_v.at[nb])
                    gather(g + 1, nb)
                else:
                    @pl.when(p < n_pairs - 1)
                    def _():
                        pltpu.sync_copy(idx_src(g + 1), idx_v.at[nb])
                        gather(g + 1, nb)
                gather_wait(b)
                write(g, b)
            return 0

        lax.fori_loop(0, n_pairs, pair_body, 0)
        write_wait(n_chunks - 1, 1)

    return k(table, idx)


def kernel(x, table):
    B0, B1 = x.shape
    D = table.shape[1]
    flat = x.reshape(B0 * B1).astype(jnp.int32)
    rows = _gather_rows(table, flat)
    return rows.reshape(B0, B1, D)
